# Initial kernel scaffold; baseline (speedup 1.0000x reference)
import jax
import jax.numpy as jnp
from jax.experimental import pallas as pl


def kernel(x, pe_table):
    raise NotImplementedError("write your pallas kernel here")


# speedup vs baseline: 1.4404x; 1.4404x over previous
import jax
import jax.numpy as jnp
from jax.experimental import pallas as pl

_B, _S, _D = 4, 8192, 768
_BS = 512


def _add_body(x_ref, pe_ref, o_ref):
    o_ref[...] = x_ref[...] + pe_ref[...]


def kernel(x, pe_table):
    grid = (_S // _BS, _B)
    return pl.pallas_call(
        _add_body,
        grid=grid,
        in_specs=[
            pl.BlockSpec((1, _BS, _D), lambda s, b: (b, s, 0)),
            pl.BlockSpec((_BS, _D), lambda s, b: (s, 0)),
        ],
        out_specs=pl.BlockSpec((1, _BS, _D), lambda s, b: (b, s, 0)),
        out_shape=jax.ShapeDtypeStruct((_B, _S, _D), jnp.float32),
    )(x, pe_table)
